# baseline (speedup 1.0000x reference)
import functools

import jax
import jax.numpy as jnp
from jax import lax
from jax.experimental import pallas as pl
from jax.experimental.pallas import tpu as pltpu
from jax.experimental.pallas import tpu_sc as plsc

B = 16384
EMB = 64
RED = 256
MAX_RATING = 5.0


_NC, _NS = 2, 16
_NW = _NC * _NS
_BPW = B // _NW


@functools.cache
def _make_gather():
    mesh = plsc.VectorSubcoreMesh(core_axis_name="c", subcore_axis_name="s",
                                  num_cores=_NC, num_subcores=_NS)

    @functools.partial(
        pl.kernel,
        mesh=mesh,
        out_type=(
            jax.ShapeDtypeStruct((B, EMB), jnp.float32),
            jax.ShapeDtypeStruct((B, EMB), jnp.float32),
        ),
        scratch_types=[
            pltpu.VMEM((_BPW,), jnp.int32),
            pltpu.VMEM((_BPW,), jnp.int32),
            pltpu.VMEM((_BPW, EMB), jnp.float32),
            pltpu.VMEM((_BPW, EMB), jnp.float32),
            pltpu.SemaphoreType.DMA,
            pltpu.SemaphoreType.DMA,
        ],
        compiler_params=pltpu.CompilerParams(use_tc_tiling_on_sc=False),
    )
    def gather_kernel(uid_hbm, fid_hbm, utab_hbm, ftab_hbm,
                      uout_hbm, fout_hbm,
                      uidx_v, fidx_v, urows_v, frows_v, usem, fsem):
        wid = lax.axis_index("s") * _NC + lax.axis_index("c")
        base = wid * _BPW
        pltpu.sync_copy(uid_hbm.at[pl.ds(base, _BPW)], uidx_v)
        pltpu.sync_copy(fid_hbm.at[pl.ds(base, _BPW)], fidx_v)
        cu = pltpu.async_copy(utab_hbm.at[uidx_v], urows_v, usem)
        cf = pltpu.async_copy(ftab_hbm.at[fidx_v], frows_v, fsem)
        cu.wait()
        cf.wait()
        pltpu.sync_copy(urows_v, uout_hbm.at[pl.ds(base, _BPW)])
        pltpu.sync_copy(frows_v, fout_hbm.at[pl.ds(base, _BPW)])

    return gather_kernel


_BLK = 2048


def _mlp_body(u_ref, f_ref, w1u_ref, w1f_ref, b1_ref, w2_ref, b2_ref,
              w3_ref, b3_ref, o_ref):
    h = jnp.dot(u_ref[...], w1u_ref[...], preferred_element_type=jnp.float32)
    h += jnp.dot(f_ref[...], w1f_ref[...], preferred_element_type=jnp.float32)
    h = jnp.maximum(h + b1_ref[...], 0.0)
    h2 = jnp.dot(h, w2_ref[...], preferred_element_type=jnp.float32)
    h2 = jnp.maximum(h2 + b2_ref[...], 0.0)
    z = jnp.sum(h2 * w3_ref[...], axis=1) + b3_ref[0]
    o_ref[...] = MAX_RATING * jax.nn.sigmoid(z)


def _mlp(u, f, w1u, w1f, b1, w2, b2, w3, b3):
    grid = (B // _BLK,)
    return pl.pallas_call(
        _mlp_body,
        grid=grid,
        in_specs=[
            pl.BlockSpec((_BLK, EMB), lambda i: (i, 0)),
            pl.BlockSpec((_BLK, EMB), lambda i: (i, 0)),
            pl.BlockSpec((EMB, RED), lambda i: (0, 0)),
            pl.BlockSpec((EMB, RED), lambda i: (0, 0)),
            pl.BlockSpec((1, RED), lambda i: (0, 0)),
            pl.BlockSpec((RED, RED // 2), lambda i: (0, 0)),
            pl.BlockSpec((1, RED // 2), lambda i: (0, 0)),
            pl.BlockSpec((1, RED // 2), lambda i: (0, 0)),
            pl.BlockSpec(memory_space=pltpu.SMEM),
        ],
        out_specs=pl.BlockSpec((_BLK,), lambda i: (i,)),
        out_shape=jax.ShapeDtypeStruct((B,), jnp.float32),
    )(u, f, w1u, w1f, b1, w2, b2, w3, b3)


@jax.jit
def kernel(user_id, film_id, user_table, film_table, W1, b1, W2, b2, W3, b3):
    user_vec, film_vec = _make_gather()(user_id.astype(jnp.int32),
                                 film_id.astype(jnp.int32),
                                 user_table, film_table)
    w1t = W1.T
    w1u = w1t[:EMB]
    w1f = w1t[EMB:]
    w2t = W2.T
    w3 = W3.reshape(1, RED // 2)
    return _mlp(user_vec, film_vec, w1u, w1f, b1.reshape(1, RED),
                w2t, b2.reshape(1, RED // 2), w3, b3)


# speedup vs baseline: 1.0073x; 1.0073x over previous
import functools

import jax
import jax.numpy as jnp
from jax import lax
from jax.experimental import pallas as pl
from jax.experimental.pallas import tpu as pltpu
from jax.experimental.pallas import tpu_sc as plsc

B = 16384
EMB = 64
RED = 256
MAX_RATING = 5.0

_NC, _NS = 2, 16
_NW = _NC * _NS
_BPW = B // _NW
_CH = 256


@functools.cache
def _make_gather():
    mesh = plsc.VectorSubcoreMesh(core_axis_name="c", subcore_axis_name="s",
                                  num_cores=_NC, num_subcores=_NS)

    @functools.partial(
        pl.kernel,
        mesh=mesh,
        out_type=(
            jax.ShapeDtypeStruct((B, 2 * EMB), jnp.float32),
            jax.ShapeDtypeStruct((B, 2 * EMB), jnp.float32),
        ),
        scratch_types=[
            pltpu.VMEM((_CH,), jnp.int32),
            pltpu.VMEM((_CH,), jnp.int32),
            pltpu.VMEM((_CH, 2 * EMB), jnp.float32),
            pltpu.VMEM((_CH, 2 * EMB), jnp.float32),
            pltpu.SemaphoreType.DMA,
            pltpu.SemaphoreType.DMA,
        ],
    )
    def gather_kernel(uid_hbm, fid_hbm, utab_hbm, ftab_hbm,
                      uout_hbm, fout_hbm,
                      uidx_v, fidx_v, urows_v, frows_v, usem, fsem):
        wid = lax.axis_index("s") * _NC + lax.axis_index("c")
        for c in range(_BPW // _CH):
            base = wid * _BPW + c * _CH
            pltpu.sync_copy(uid_hbm.at[pl.ds(base, _CH)], uidx_v)
            pltpu.sync_copy(fid_hbm.at[pl.ds(base, _CH)], fidx_v)
            cu = pltpu.async_copy(utab_hbm.at[uidx_v], urows_v, usem)
            cf = pltpu.async_copy(ftab_hbm.at[fidx_v], frows_v, fsem)
            cu.wait()
            cf.wait()
            pltpu.sync_copy(urows_v, uout_hbm.at[pl.ds(base, _CH)])
            pltpu.sync_copy(frows_v, fout_hbm.at[pl.ds(base, _CH)])

    return gather_kernel


_BLK = 2048


def _mlp_body(upar_ref, fpar_ref, upair_ref, fpair_ref,
              w1u_ref, w1f_ref, b1_ref, w2_ref, b2_ref,
              w3_ref, b3_ref, o_ref):
    u = jnp.where(upar_ref[...] == 0,
                  upair_ref[:, :EMB], upair_ref[:, EMB:])
    f = jnp.where(fpar_ref[...] == 0,
                  fpair_ref[:, :EMB], fpair_ref[:, EMB:])
    h = jnp.dot(u, w1u_ref[...], preferred_element_type=jnp.float32)
    h += jnp.dot(f, w1f_ref[...], preferred_element_type=jnp.float32)
    h = jnp.maximum(h + b1_ref[...], 0.0)
    h2 = jnp.dot(h, w2_ref[...], preferred_element_type=jnp.float32)
    h2 = jnp.maximum(h2 + b2_ref[...], 0.0)
    z = jnp.sum(h2 * w3_ref[...], axis=1) + b3_ref[0]
    o_ref[...] = MAX_RATING * jax.nn.sigmoid(z)


def _mlp(upar, fpar, upair, fpair, w1u, w1f, b1, w2, b2, w3, b3):
    grid = (B // _BLK,)
    return pl.pallas_call(
        _mlp_body,
        grid=grid,
        in_specs=[
            pl.BlockSpec((_BLK, 1), lambda i: (i, 0)),
            pl.BlockSpec((_BLK, 1), lambda i: (i, 0)),
            pl.BlockSpec((_BLK, 2 * EMB), lambda i: (i, 0)),
            pl.BlockSpec((_BLK, 2 * EMB), lambda i: (i, 0)),
            pl.BlockSpec((EMB, RED), lambda i: (0, 0)),
            pl.BlockSpec((EMB, RED), lambda i: (0, 0)),
            pl.BlockSpec((1, RED), lambda i: (0, 0)),
            pl.BlockSpec((RED, RED // 2), lambda i: (0, 0)),
            pl.BlockSpec((1, RED // 2), lambda i: (0, 0)),
            pl.BlockSpec((1, RED // 2), lambda i: (0, 0)),
            pl.BlockSpec(memory_space=pltpu.SMEM),
        ],
        out_specs=pl.BlockSpec((_BLK,), lambda i: (i,)),
        out_shape=jax.ShapeDtypeStruct((B,), jnp.float32),
    )(upar, fpar, upair, fpair, w1u, w1f, b1, w2, b2, w3, b3)


@jax.jit
def kernel(user_id, film_id, user_table, film_table, W1, b1, W2, b2, W3, b3):
    uid = user_id.astype(jnp.int32)
    fid = film_id.astype(jnp.int32)
    utab2 = user_table.reshape(-1, 2 * EMB)
    ftab2 = film_table.reshape(-1, 2 * EMB)
    upair, fpair = _make_gather()(uid >> 1, fid >> 1, utab2, ftab2)
    upar = (uid & 1).reshape(B, 1)
    fpar = (fid & 1).reshape(B, 1)
    w1t = W1.T
    w1u = w1t[:EMB]
    w1f = w1t[EMB:]
    w2t = W2.T
    w3 = W3.reshape(1, RED // 2)
    return _mlp(upar, fpar, upair, fpair, w1u, w1f, b1.reshape(1, RED),
                w2t, b2.reshape(1, RED // 2), w3, b3)


# speedup vs baseline: 1.3817x; 1.3718x over previous
import functools

import jax
import jax.numpy as jnp
from jax import lax
from jax.experimental import pallas as pl
from jax.experimental.pallas import tpu as pltpu
from jax.experimental.pallas import tpu_sc as plsc

B = 16384
EMB = 64
RED = 256
MAX_RATING = 5.0

_NC, _NS = 2, 16
_NW = _NC * _NS
_CH = B // _NW
_G = 16


@functools.cache
def _make_gather():
    mesh = plsc.VectorSubcoreMesh(core_axis_name="c", subcore_axis_name="s",
                                  num_cores=_NC, num_subcores=_NS)

    @functools.partial(
        pl.kernel,
        mesh=mesh,
        out_type=(
            jax.ShapeDtypeStruct((B * EMB,), jnp.float32),
            jax.ShapeDtypeStruct((B * EMB,), jnp.float32),
        ),
        scratch_types=[
            pltpu.VMEM((_CH,), jnp.int32),
            pltpu.VMEM((_CH,), jnp.int32),
            pltpu.VMEM((8 * _G, EMB), jnp.float32),
            pltpu.VMEM((8 * _G, EMB), jnp.float32),
            pltpu.SemaphoreType.DMA,
            pltpu.SemaphoreType.DMA,
            pltpu.SemaphoreType.DMA,
            pltpu.SemaphoreType.DMA,
        ],
    )
    def gather_kernel(uid_hbm, fid_hbm, utab_hbm, ftab_hbm,
                      uout_hbm, fout_hbm,
                      uidx_v, fidx_v, ubuf_v, fbuf_v,
                      gsem_u, gsem_f, wsem_u, wsem_f):
        wid = lax.axis_index("s") * _NC + lax.axis_index("c")
        base = wid * _CH
        pltpu.sync_copy(uid_hbm.at[pl.ds(base, _CH)], uidx_v)
        pltpu.sync_copy(fid_hbm.at[pl.ds(base, _CH)], fidx_v)

        def group(g, carry):
            goff = pl.multiple_of(g * _G, _G)
            uv = uidx_v[pl.ds(goff, _G)]
            fv = fidx_v[pl.ds(goff, _G)]
            gh = []
            for j in range(_G):
                urow = pl.multiple_of((uv[j] >> 3) * 8, 8)
                frow = pl.multiple_of((fv[j] >> 3) * 8, 8)
                gh.append(pltpu.async_copy(
                    utab_hbm.at[pl.ds(urow, 8), :],
                    ubuf_v.at[pl.ds(8 * j, 8), :], gsem_u))
                gh.append(pltpu.async_copy(
                    ftab_hbm.at[pl.ds(frow, 8), :],
                    fbuf_v.at[pl.ds(8 * j, 8), :], gsem_f))
            for h in gh:
                h.wait()
            wh = []
            for j in range(_G):
                out_off = pl.multiple_of((base + goff + j) * EMB, EMB)
                wh.append(pltpu.async_copy(
                    ubuf_v.at[8 * j + (uv[j] & 7), :],
                    uout_hbm.at[pl.ds(out_off, EMB)], wsem_u))
                wh.append(pltpu.async_copy(
                    fbuf_v.at[8 * j + (fv[j] & 7), :],
                    fout_hbm.at[pl.ds(out_off, EMB)], wsem_f))
            for h in wh:
                h.wait()
            return carry

        lax.fori_loop(0, _CH // _G, group, 0)

    return gather_kernel


_BLK = 2048


def _mlp_body(u_ref, f_ref, w1u_ref, w1f_ref, b1_ref, w2_ref, b2_ref,
              w3_ref, b3_ref, o_ref):
    h = jnp.dot(u_ref[...], w1u_ref[...], preferred_element_type=jnp.float32)
    h += jnp.dot(f_ref[...], w1f_ref[...], preferred_element_type=jnp.float32)
    h = jnp.maximum(h + b1_ref[...], 0.0)
    h2 = jnp.dot(h, w2_ref[...], preferred_element_type=jnp.float32)
    h2 = jnp.maximum(h2 + b2_ref[...], 0.0)
    z = jnp.sum(h2 * w3_ref[...], axis=1) + b3_ref[0]
    o_ref[...] = MAX_RATING * jax.nn.sigmoid(z)


def _mlp(u, f, w1u, w1f, b1, w2, b2, w3, b3):
    grid = (B // _BLK,)
    return pl.pallas_call(
        _mlp_body,
        grid=grid,
        in_specs=[
            pl.BlockSpec((_BLK, EMB), lambda i: (i, 0)),
            pl.BlockSpec((_BLK, EMB), lambda i: (i, 0)),
            pl.BlockSpec((EMB, RED), lambda i: (0, 0)),
            pl.BlockSpec((EMB, RED), lambda i: (0, 0)),
            pl.BlockSpec((1, RED), lambda i: (0, 0)),
            pl.BlockSpec((RED, RED // 2), lambda i: (0, 0)),
            pl.BlockSpec((1, RED // 2), lambda i: (0, 0)),
            pl.BlockSpec((1, RED // 2), lambda i: (0, 0)),
            pl.BlockSpec(memory_space=pltpu.SMEM),
        ],
        out_specs=pl.BlockSpec((_BLK,), lambda i: (i,)),
        out_shape=jax.ShapeDtypeStruct((B,), jnp.float32),
    )(u, f, w1u, w1f, b1, w2, b2, w3, b3)


@jax.jit
def kernel(user_id, film_id, user_table, film_table, W1, b1, W2, b2, W3, b3):
    uid = user_id.astype(jnp.int32)
    fid = film_id.astype(jnp.int32)
    uflat, fflat = _make_gather()(uid, fid, user_table, film_table)
    u = uflat.reshape(B, EMB)
    f = fflat.reshape(B, EMB)
    w1t = W1.T
    w1u = w1t[:EMB]
    w1f = w1t[EMB:]
    w2t = W2.T
    w3 = W3.reshape(1, RED // 2)
    return _mlp(u, f, w1u, w1f, b1.reshape(1, RED),
                w2t, b2.reshape(1, RED // 2), w3, b3)
